# baseline (speedup 1.0000x reference)
import math

import jax
import jax.numpy as jnp
from jax import lax
from jax.experimental import pallas as pl
from jax.experimental.pallas import tpu as pltpu
from jax.experimental.pallas import tpu_sc as plsc

_INFO = plsc.get_sparse_core_info()
_NC = _INFO.num_cores
_NS = _INFO.num_subcores
_NW = _NC * _NS

_CHUNK = 8
_NBUF = 3


def _make_sc_kernel(B, S, V, D):
    assert S % _NW == 0
    seq_per_w = S // _NW
    chunk = _CHUNK
    nbuf = _NBUF
    assert seq_per_w % chunk == 0
    nchunk = seq_per_w // chunk
    rows_per_chunk = B * chunk
    scale = float(math.sqrt(D))
    jblocks = D // 16
    jshift = jblocks.bit_length() - 1
    scratch = (
        [pltpu.VMEM((rows_per_chunk,), jnp.int32)] * nchunk
        + [pltpu.VMEM((rows_per_chunk, D), jnp.float32)] * nbuf
        + [pltpu.VMEM((chunk, D), jnp.float32)] * nbuf
        + [pltpu.SemaphoreType.DMA] * (3 * nbuf + 1)
    )
    mesh = plsc.VectorSubcoreMesh(core_axis_name="c", subcore_axis_name="s")

    def body(tok_perm_hbm, table_hbm, pe_hbm, out_hbm, *bufs):
        idx_refs = bufs[:nchunk]
        bufs = bufs[nchunk:]
        rows = bufs[:nbuf]
        peb = bufs[nbuf:2 * nbuf]
        gsem = bufs[2 * nbuf:3 * nbuf]
        psem = bufs[3 * nbuf:4 * nbuf]
        osem = bufs[4 * nbuf:5 * nbuf]
        isem = bufs[5 * nbuf]
        wid = lax.axis_index("s") * _NC + lax.axis_index("c")
        sbase0 = pl.multiple_of(wid * seq_per_w, seq_per_w)

        idx_cps = [
            pltpu.async_copy(tok_perm_hbm.at[wid, p], idx_refs[p], isem)
            for p in range(nchunk)
        ]

        def start_in(p):
            s = p % nbuf
            g = pltpu.async_copy(table_hbm.at[idx_refs[p]], rows[s], gsem[s])
            q = pltpu.async_copy(
                pe_hbm.at[pl.ds(pl.multiple_of(sbase0 + p * chunk, chunk),
                                chunk)],
                peb[s], psem[s])
            return g, q

        def start_out(p):
            s = p % nbuf
            return [
                pltpu.async_copy(
                    rows[s].at[pl.ds(b * chunk, chunk)],
                    out_hbm.at[pl.ds(
                        pl.multiple_of(b * S + sbase0 + p * chunk, chunk),
                        chunk)],
                    osem[s])
                for b in range(B)
            ]

        depth = nbuf - 1
        for cp in idx_cps:
            cp.wait()
        pending_in = {p: start_in(p) for p in range(min(depth, nchunk))}
        pending_out = {}
        for p in range(nchunk):
            s = p % nbuf
            g, q = pending_in.pop(p)
            g.wait()
            q.wait()

            @plsc.parallel_loop(0, chunk * jblocks, unroll=4)
            def _(i):
                r = i >> jshift
                o = pl.multiple_of((i & (jblocks - 1)) * 16, 16)
                pec = peb[s][r, pl.ds(o, 16)]
                for b in range(B):
                    rows[s][b * chunk + r, pl.ds(o, 16)] = (
                        rows[s][b * chunk + r, pl.ds(o, 16)] * scale + pec)

            pending_out[p] = start_out(p)
            nxt = p + depth
            if nxt < nchunk:
                prev = nxt - nbuf
                if prev in pending_out:
                    for cp in pending_out.pop(prev):
                        cp.wait()
                pending_in[nxt] = start_in(nxt)
        for p in sorted(pending_out):
            for cp in pending_out.pop(p):
                cp.wait()

    return pl.kernel(
        body,
        out_type=jax.ShapeDtypeStruct((B * S, D), jnp.float32),
        mesh=mesh,
        scratch_types=scratch,
        compiler_params=pltpu.CompilerParams(use_tc_tiling_on_sc=False),
    )


def kernel(token_tensor, emb_table, pe):
    B, S = token_tensor.shape
    V, D = emb_table.shape
    seq_per_w = S // _NW
    nchunk = seq_per_w // _CHUNK
    tok_perm = (token_tensor.astype(jnp.int32)
                .reshape(B, _NW, nchunk, _CHUNK)
                .transpose(1, 2, 0, 3)
                .reshape(_NW, nchunk, B * _CHUNK))
    out = _make_sc_kernel(B, S, V, D)(tok_perm, emb_table, pe)
    return out.reshape(B, S, D)


# speedup vs baseline: 7.4160x; 7.4160x over previous
import math

import jax
import jax.numpy as jnp
from jax import lax
from jax.experimental import pallas as pl
from jax.experimental.pallas import tpu as pltpu
from jax.experimental.pallas import tpu_sc as plsc

_INFO = plsc.get_sparse_core_info()
_NC = _INFO.num_cores
_NS = _INFO.num_subcores
_NW = _NC * _NS

_CHUNK = 8
_NBUF = 3


def _make_sc_kernel(B, S, V, D):
    assert S % _NW == 0
    seq_per_w = S // _NW
    chunk = _CHUNK
    nbuf = _NBUF
    assert seq_per_w % chunk == 0
    nchunk = seq_per_w // chunk
    rows_per_chunk = B * chunk
    scale = float(math.sqrt(D))
    jblocks = D // 16
    jshift = jblocks.bit_length() - 1
    scratch = (
        [pltpu.VMEM((rows_per_chunk,), jnp.int32)] * nchunk
        + [pltpu.VMEM((rows_per_chunk, D), jnp.float32)] * nbuf
        + [pltpu.VMEM((chunk, D), jnp.float32)] * nbuf
        + [pltpu.SemaphoreType.DMA] * (3 * nbuf + 1)
    )
    mesh = plsc.VectorSubcoreMesh(core_axis_name="c", subcore_axis_name="s")

    def body(tok_perm_hbm, table_hbm, pe_hbm, out_hbm, *bufs):
        idx_refs = bufs[:nchunk]
        bufs = bufs[nchunk:]
        rows = bufs[:nbuf]
        peb = bufs[nbuf:2 * nbuf]
        gsem = bufs[2 * nbuf:3 * nbuf]
        psem = bufs[3 * nbuf:4 * nbuf]
        osem = bufs[4 * nbuf:5 * nbuf]
        isem = bufs[5 * nbuf]
        wid = lax.axis_index("s") * _NC + lax.axis_index("c")
        sbase0 = pl.multiple_of(wid * seq_per_w, seq_per_w)

        idx_cps = [
            pltpu.async_copy(tok_perm_hbm.at[wid, p], idx_refs[p], isem)
            for p in range(nchunk)
        ]

        def start_in(p):
            s = p % nbuf
            g = pltpu.async_copy(table_hbm.at[idx_refs[p]], rows[s], gsem[s])
            q = pltpu.async_copy(
                pe_hbm.at[pl.ds(pl.multiple_of(sbase0 + p * chunk, chunk),
                                chunk)],
                peb[s], psem[s])
            return g, q

        def start_out(p):
            s = p % nbuf
            return [
                pltpu.async_copy(
                    rows[s].at[pl.ds(b * chunk, chunk)],
                    out_hbm.at[pl.ds(
                        pl.multiple_of(b * S + sbase0 + p * chunk, chunk),
                        chunk)],
                    osem[s])
                for b in range(B)
            ]

        depth = nbuf - 1
        pending_in = {}
        for p in range(min(depth, nchunk)):
            idx_cps[p].wait()
            pending_in[p] = start_in(p)
        pending_out = {}
        for p in range(nchunk):
            s = p % nbuf
            g, q = pending_in.pop(p)
            g.wait()
            q.wait()

            @plsc.parallel_loop(0, chunk * jblocks, unroll=8)
            def _(i):
                r = i >> jshift
                o = pl.multiple_of((i & (jblocks - 1)) * 16, 16)
                pec = peb[s][r, pl.ds(o, 16)]
                for b in range(B):
                    rows[s][b * chunk + r, pl.ds(o, 16)] = (
                        rows[s][b * chunk + r, pl.ds(o, 16)] * scale + pec)

            pending_out[p] = start_out(p)
            nxt = p + depth
            if nxt < nchunk:
                prev = nxt - nbuf
                if prev in pending_out:
                    for cp in pending_out.pop(prev):
                        cp.wait()
                idx_cps[nxt].wait()
                pending_in[nxt] = start_in(nxt)
        for p in sorted(pending_out):
            for cp in pending_out.pop(p):
                cp.wait()

    return pl.kernel(
        body,
        out_type=jax.ShapeDtypeStruct((B * S, D), jnp.float32),
        mesh=mesh,
        scratch_types=scratch,
    )


def kernel(token_tensor, emb_table, pe):
    B, S = token_tensor.shape
    V, D = emb_table.shape
    seq_per_w = S // _NW
    nchunk = seq_per_w // _CHUNK
    tok_perm = (token_tensor.astype(jnp.int32)
                .reshape(B, _NW, nchunk, _CHUNK)
                .transpose(1, 2, 0, 3)
                .reshape(_NW, nchunk, B * _CHUNK))
    out = _make_sc_kernel(B, S, V, D)(tok_perm, emb_table, pe)
    return out.reshape(B, S, D)
